# baseline (speedup 1.0000x reference)
import functools

import jax
import jax.numpy as jnp
from jax import lax
from jax.experimental import pallas as pl
from jax.experimental.pallas import tpu as pltpu
from jax.experimental.pallas import tpu_sc as plsc


def _make_kernel(B, L, V, E):
    info = plsc.get_sparse_core_info()
    NC, NS, LANES = info.num_cores, info.num_subcores, info.num_lanes
    NW = NC * NS
    assert B % NW == 0
    seq_per_w = B // NW
    vecs_per_row = E // LANES

    mesh = plsc.VectorSubcoreMesh(core_axis_name="c", subcore_axis_name="s")

    @functools.partial(
        pl.kernel,
        mesh=mesh,
        out_type=jax.ShapeDtypeStruct((B * L, E), jnp.float32),
        scratch_types=[
            pltpu.VMEM((L, E), jnp.float32),
            pltpu.VMEM((L,), jnp.int32),
            pltpu.VMEM((L, E), jnp.float32),
            pltpu.SemaphoreType.DMA,
        ],
        compiler_params=pltpu.CompilerParams(use_tc_tiling_on_sc=False),
    )
    def k(tok_hbm, idx_hbm, pos_hbm, out_hbm, pos_v, idx_v, rows_v, sem):
        wid = lax.axis_index("s") * NC + lax.axis_index("c")
        pltpu.sync_copy(pos_hbm, pos_v)

        def seq_body(s, carry):
            base = (wid * seq_per_w + s) * L
            pltpu.sync_copy(idx_hbm.at[pl.ds(base, L)], idx_v)
            cp0 = pltpu.make_async_copy(
                tok_hbm.at[idx_v.at[pl.ds(0, 128)]], rows_v.at[pl.ds(0, 128)], sem)
            cp1 = pltpu.make_async_copy(
                tok_hbm.at[idx_v.at[pl.ds(128, L - 128)]],
                rows_v.at[pl.ds(128, L - 128)], sem)
            cp0.start()
            cp1.start()
            cp0.wait()
            cp1.wait()

            def add_row(r, carry2):
                for c in range(vecs_per_row):
                    sl = pl.ds(c * LANES, LANES)
                    rows_v[r, sl] = rows_v[r, sl] + pos_v[r, sl]
                return carry2

            lax.fori_loop(0, L, add_row, 0, unroll=4)
            pltpu.sync_copy(rows_v, out_hbm.at[pl.ds(base, L)])
            return carry

        lax.fori_loop(0, seq_per_w, seq_body, 0)

    return k


def kernel(x, tok_table, pos_table):
    B, L = x.shape
    V, E = tok_table.shape
    k = _make_kernel(B, L, V, E)
    out_flat = k(tok_table, x.reshape(-1), pos_table)
    return out_flat.reshape(B, L, E)


# speedup vs baseline: 1.4438x; 1.4438x over previous
import functools

import jax
import jax.numpy as jnp
from jax import lax
from jax.experimental import pallas as pl
from jax.experimental.pallas import tpu as pltpu
from jax.experimental.pallas import tpu_sc as plsc


def _make_kernel(B, L, V, E):
    info = plsc.get_sparse_core_info()
    NC, NS, LANES = info.num_cores, info.num_subcores, info.num_lanes
    NW = NC * NS
    SPC = 2
    CH = SPC * L
    assert B % (NW * SPC) == 0 and E % LANES == 0
    nchunks = B // (NW * SPC)
    gchunks = []
    off = 0
    while off < CH:
        sz = min(128, CH - off)
        gchunks.append((off, sz))
        off += sz

    mesh = plsc.VectorSubcoreMesh(core_axis_name="c", subcore_axis_name="s")

    @functools.partial(
        pl.kernel,
        mesh=mesh,
        out_type=jax.ShapeDtypeStruct((B * L, E), jnp.float32),
        scratch_types=[
            pltpu.VMEM((CH, E), jnp.float32),
            [pltpu.VMEM((CH,), jnp.int32)] * 2,
            [pltpu.VMEM((CH, E), jnp.float32)] * 2,
            [pltpu.SemaphoreType.DMA] * 6,
        ],
        compiler_params=pltpu.CompilerParams(use_tc_tiling_on_sc=False),
    )
    def k(tok_hbm, idx_hbm, pos_hbm, out_hbm, pos_v, idx_v, rows_v, sems):
        wid = lax.axis_index("s") * NC + lax.axis_index("c")
        gsem, isem, wsem = sems[0:2], sems[2:4], sems[4:6]
        for r in range(SPC):
            pltpu.sync_copy(pos_hbm, pos_v.at[pl.ds(r * L, L)])
        base0 = wid * nchunks * CH

        def stage_idx(c, b):
            pltpu.make_async_copy(
                idx_hbm.at[pl.ds(base0 + c * CH, CH)], idx_v[b], isem[b]).start()

        def iwait(b):
            pltpu.make_async_copy(
                idx_hbm.at[pl.ds(0, CH)], idx_v[b], isem[b]).wait()

        def start_gathers(b):
            for goff, gsz in gchunks:
                pltpu.make_async_copy(
                    tok_hbm.at[idx_v[b].at[pl.ds(goff, gsz)]],
                    rows_v[b].at[pl.ds(goff, gsz)], gsem[b]).start()

        def gwait(b):
            pltpu.make_async_copy(
                tok_hbm.at[pl.ds(0, CH)], rows_v[b], gsem[b]).wait()

        def start_wb(c, b):
            pltpu.make_async_copy(
                rows_v[b], out_hbm.at[pl.ds(base0 + c * CH, CH)], wsem[b]).start()

        def wwait(b):
            pltpu.make_async_copy(
                rows_v[b], out_hbm.at[pl.ds(0, CH)], wsem[b]).wait()

        def add_pos(b):
            rows = rows_v[b]

            def add_row(r, carry):
                for v in range(E // LANES):
                    sl = pl.ds(v * LANES, LANES)
                    plsc.addupdate(rows.at[r, sl], pos_v[r, sl])
                return carry

            lax.fori_loop(0, CH, add_row, 0, unroll=4)

        def step(c, b, *, wwait_nb, gather_next, stage_next):
            nb = 1 - b
            gwait(b)
            if gather_next:
                iwait(nb)
                if wwait_nb:
                    wwait(nb)
                start_gathers(nb)
            if stage_next:
                stage_idx(c + 2, b)
            add_pos(b)
            start_wb(c, b)

        stage_idx(0, 0)
        stage_idx(1, 1)
        iwait(0)
        start_gathers(0)
        step(0, 0, wwait_nb=False, gather_next=True, stage_next=True)
        step(1, 1, wwait_nb=True, gather_next=True, stage_next=True)

        def loop_body(i, carry):
            c = 2 + 2 * i
            step(c, 0, wwait_nb=True, gather_next=True, stage_next=True)
            step(c + 1, 1, wwait_nb=True, gather_next=True, stage_next=True)
            return carry

        lax.fori_loop(0, (nchunks - 4) // 2, loop_body, 0)
        step(nchunks - 2, 0, wwait_nb=True, gather_next=True, stage_next=False)
        step(nchunks - 1, 1, wwait_nb=False, gather_next=False, stage_next=False)
        wwait(0)
        wwait(1)

    return k


def kernel(x, tok_table, pos_table):
    B, L = x.shape
    V, E = tok_table.shape
    k = _make_kernel(B, L, V, E)
    out_flat = k(tok_table, x.reshape(-1), pos_table)
    return out_flat.reshape(B, L, E)
